# baseline (speedup 1.0000x reference)
import jax
import jax.numpy as jnp
from jax.experimental import pallas as pl

N = 512
U = 64
B = 64
NK = 3


def _prep_kernel(adj_ref, x_ref,
                 wru0_ref, wc0_ref, wru1_ref, wc1_ref,
                 bru0_ref, bc0_ref, bru1_ref, bc1_ref, wp_ref,
                 sup_ref, a1_ref, a2_ref,
                 wa_ru0_ref, wh_ru0_ref, b_ru0_ref,
                 wa_c0_ref, wh_c0_ref, b_c0_ref,
                 wg_ru1_ref, wk_ru1_ref, b_ru1_ref,
                 wg_c1_ref, wk_c1_ref, b_c1_ref, wpp_ref):
    f32 = jnp.float32
    adj = adj_ref[...]
    a = jnp.maximum(adj, adj.T)
    d_col = jnp.sum(a, axis=1, keepdims=True)
    d_row = jnp.sum(a, axis=0, keepdims=True)
    inv_c = jnp.where(d_col > 0, 1.0 / jnp.sqrt(d_col), 0.0)
    inv_r = jnp.where(d_row > 0, 1.0 / jnp.sqrt(d_row), 0.0)
    sup = -(inv_c * a) * inv_r
    sup_ref[...] = sup
    x0 = x_ref[...]
    a1 = jnp.dot(x0, sup, preferred_element_type=f32)
    a1_ref[...] = a1
    a2_ref[...] = 2.0 * jnp.dot(a1, sup, preferred_element_type=f32) - x0

    wa_ru0_ref[...] = jnp.zeros((2 * NK, 4 * U), f32)
    wa_c0_ref[...] = jnp.zeros((2 * NK, 2 * U), f32)
    wh_ru0_ref[...] = jnp.zeros((NK, 2 * U, 4 * U), f32)
    wh_c0_ref[...] = jnp.zeros((NK, 2 * U, 2 * U), f32)
    wg_ru1_ref[...] = jnp.zeros((NK, 2 * U, 4 * U), f32)
    wk_ru1_ref[...] = jnp.zeros((NK, 2 * U, 4 * U), f32)
    wg_c1_ref[...] = jnp.zeros((NK, 2 * U, 2 * U), f32)
    wk_c1_ref[...] = jnp.zeros((NK, 2 * U, 2 * U), f32)
    for k in range(NK):
        wa_ru0_ref[2 * k, 0:U] = wru0_ref[k, 0, 0:U]
        wa_ru0_ref[2 * k, 2 * U:3 * U] = wru0_ref[k, 0, U:2 * U]
        wa_ru0_ref[2 * k + 1, U:2 * U] = wru0_ref[k, 0, 0:U]
        wa_ru0_ref[2 * k + 1, 3 * U:4 * U] = wru0_ref[k, 0, U:2 * U]
        wa_c0_ref[2 * k, 0:U] = wc0_ref[k, 0, :]
        wa_c0_ref[2 * k + 1, U:2 * U] = wc0_ref[k, 0, :]
        whr = wru0_ref[k, 1:U + 1, 0:U]
        whu = wru0_ref[k, 1:U + 1, U:2 * U]
        wh_ru0_ref[k, 0:U, 0:U] = whr
        wh_ru0_ref[k, 0:U, 2 * U:3 * U] = whu
        wh_ru0_ref[k, U:2 * U, U:2 * U] = whr
        wh_ru0_ref[k, U:2 * U, 3 * U:4 * U] = whu
        whc = wc0_ref[k, 1:U + 1, :]
        wh_c0_ref[k, 0:U, 0:U] = whc
        wh_c0_ref[k, U:2 * U, U:2 * U] = whc
        ggr = wru1_ref[k, 0:U, 0:U]
        ggu = wru1_ref[k, 0:U, U:2 * U]
        wg_ru1_ref[k, 0:U, 0:U] = ggr
        wg_ru1_ref[k, 0:U, 2 * U:3 * U] = ggu
        wg_ru1_ref[k, U:2 * U, U:2 * U] = ggr
        wg_ru1_ref[k, U:2 * U, 3 * U:4 * U] = ggu
        kkr = wru1_ref[k, U:2 * U, 0:U]
        kku = wru1_ref[k, U:2 * U, U:2 * U]
        wk_ru1_ref[k, 0:U, 0:U] = kkr
        wk_ru1_ref[k, 0:U, 2 * U:3 * U] = kku
        wk_ru1_ref[k, U:2 * U, U:2 * U] = kkr
        wk_ru1_ref[k, U:2 * U, 3 * U:4 * U] = kku
        ggc = wc1_ref[k, 0:U, :]
        wg_c1_ref[k, 0:U, 0:U] = ggc
        wg_c1_ref[k, U:2 * U, U:2 * U] = ggc
        kkc = wc1_ref[k, U:2 * U, :]
        wk_c1_ref[k, 0:U, 0:U] = kkc
        wk_c1_ref[k, U:2 * U, U:2 * U] = kkc

    b_ru0_ref[0, 0:U] = bru0_ref[0, 0:U]
    b_ru0_ref[0, U:2 * U] = bru0_ref[0, 0:U]
    b_ru0_ref[0, 2 * U:3 * U] = bru0_ref[0, U:2 * U]
    b_ru0_ref[0, 3 * U:4 * U] = bru0_ref[0, U:2 * U]
    b_c0_ref[0, 0:U] = bc0_ref[0, :]
    b_c0_ref[0, U:2 * U] = bc0_ref[0, :]
    b_ru1_ref[0, 0:U] = bru1_ref[0, 0:U]
    b_ru1_ref[0, U:2 * U] = bru1_ref[0, 0:U]
    b_ru1_ref[0, 2 * U:3 * U] = bru1_ref[0, U:2 * U]
    b_ru1_ref[0, 3 * U:4 * U] = bru1_ref[0, U:2 * U]
    b_c1_ref[0, 0:U] = bc1_ref[0, :]
    b_c1_ref[0, U:2 * U] = bc1_ref[0, :]
    wpp_ref[...] = jnp.zeros((2 * U, 2), f32)
    wpp_ref[0:U, 0:1] = wp_ref[...]
    wpp_ref[U:2 * U, 1:2] = wp_ref[...]


def _main_kernel(sup_ref, ac_ref, h_ref,
                 wa_ru0_ref, wh_ru0_ref, b_ru0_ref,
                 wa_c0_ref, wh_c0_ref, b_c0_ref,
                 wg_ru1_ref, wk_ru1_ref, b_ru1_ref,
                 wg_c1_ref, wk_c1_ref, b_c1_ref,
                 wp_ref, bp_ref,
                 out_ref, hid_ref):
    f32 = jnp.float32
    dot = lambda x, y: jnp.dot(x, y, preferred_element_type=f32)
    S = sup_ref[...]
    A = jnp.concatenate([ac_ref[0, 0], ac_ref[1, 0], ac_ref[2, 0]],
                        axis=0).T

    H0 = jnp.concatenate([h_ref[0, 0], h_ref[0, 1]], axis=1)
    K0 = jnp.concatenate([h_ref[1, 0], h_ref[1, 1]], axis=1)
    HK0 = jnp.concatenate([H0, K0], axis=1)
    HK1 = dot(S, HK0)
    H1 = HK1[:, :2 * U]
    K1 = HK1[:, 2 * U:]
    HK2 = 2.0 * dot(S, HK1) - HK0
    H2 = HK2[:, :2 * U]
    K2 = HK2[:, 2 * U:]
    ru = b_ru0_ref[...] + dot(A, wa_ru0_ref[...])
    ru += dot(H0, wh_ru0_ref[0])
    ru += dot(H1, wh_ru0_ref[1])
    ru += dot(H2, wh_ru0_ref[2])
    val = jax.nn.sigmoid(ru)
    r = val[:, :2 * U]
    u = val[:, 2 * U:]
    rh = r * H0
    c = b_c0_ref[...] + dot(A, wa_c0_ref[...])
    c += dot(rh, wh_c0_ref[0])
    R1 = dot(S, rh)
    c += dot(R1, wh_c0_ref[1])
    R2 = 2.0 * dot(S, R1) - rh
    c += dot(R2, wh_c0_ref[2])
    c = jnp.tanh(c)
    h0n = u * H0 + (1.0 - u) * c
    hid_ref[0, 0] = h0n[:, :U]
    hid_ref[0, 1] = h0n[:, U:]

    ru1 = b_ru1_ref[...] + dot(h0n, wg_ru1_ref[0]) + dot(K0, wk_ru1_ref[0])
    ru1 += dot(K1, wk_ru1_ref[1]) + dot(K2, wk_ru1_ref[2])
    G1 = dot(S, h0n)
    ru1 += dot(G1, wg_ru1_ref[1])
    G2 = 2.0 * dot(S, G1) - h0n
    ru1 += dot(G2, wg_ru1_ref[2])
    v1 = jax.nn.sigmoid(ru1)
    r1 = v1[:, :2 * U]
    u1 = v1[:, 2 * U:]
    rh1 = r1 * K0
    c1 = (b_c1_ref[...] + dot(h0n, wg_c1_ref[0]) + dot(G1, wg_c1_ref[1])
          + dot(G2, wg_c1_ref[2]) + dot(rh1, wk_c1_ref[0]))
    Q1 = dot(S, rh1)
    c1 += dot(Q1, wk_c1_ref[1])
    Q2 = 2.0 * dot(S, Q1) - rh1
    c1 += dot(Q2, wk_c1_ref[2])
    c1 = jnp.tanh(c1)
    h1n = u1 * K0 + (1.0 - u1) * c1
    hid_ref[1, 0] = h1n[:, :U]
    hid_ref[1, 1] = h1n[:, U:]
    prj = dot(h1n, wp_ref[...]) + bp_ref[...]
    out_ref[0] = prj.T


def kernel(inputs, hidden_state, adj_mx, W_ru_0, b_ru_0, W_c_0, b_c_0,
           W_ru_1, b_ru_1, W_c_1, b_c_1, W_proj, b_proj):
    f32 = jnp.float32

    wru0 = W_ru_0.reshape(U + 1, NK, 2 * U).transpose(1, 0, 2)
    wc0 = W_c_0.reshape(U + 1, NK, U).transpose(1, 0, 2)
    wru1 = W_ru_1.reshape(2 * U, NK, 2 * U).transpose(1, 0, 2)
    wc1 = W_c_1.reshape(2 * U, NK, U).transpose(1, 0, 2)

    sds = jax.ShapeDtypeStruct
    (support, A1, A2,
     wa_ru0, wh_ru0, b_ru0, wa_c0, wh_c0, b_c0,
     wg_ru1, wk_ru1, b_ru1, wg_c1, wk_c1, b_c1, wpp) = pl.pallas_call(
        _prep_kernel,
        out_shape=[
            sds((N, N), f32), sds((B, N), f32), sds((B, N), f32),
            sds((2 * NK, 4 * U), f32), sds((NK, 2 * U, 4 * U), f32), sds((1, 4 * U), f32),
            sds((2 * NK, 2 * U), f32), sds((NK, 2 * U, 2 * U), f32), sds((1, 2 * U), f32),
            sds((NK, 2 * U, 4 * U), f32), sds((NK, 2 * U, 4 * U), f32), sds((1, 4 * U), f32),
            sds((NK, 2 * U, 2 * U), f32), sds((NK, 2 * U, 2 * U), f32), sds((1, 2 * U), f32),
            sds((2 * U, 2), f32),
        ],
    )(adj_mx, inputs, wru0, wc0, wru1, wc1,
      b_ru_0.reshape(1, 2 * U), b_c_0.reshape(1, U),
      b_ru_1.reshape(1, 2 * U), b_c_1.reshape(1, U), W_proj)

    full = lambda shape: pl.BlockSpec(shape, lambda b: tuple(0 for _ in shape))
    acol = pl.BlockSpec((NK, 1, 2, N), lambda b: (0, b, 0, 0))
    ocol = pl.BlockSpec((1, 2, N), lambda b: (b, 0, 0))
    hblk = pl.BlockSpec((2, 2, N, U), lambda b: (0, b, 0, 0))

    out_p, hid = pl.pallas_call(
        _main_kernel,
        grid=(B // 2,),
        in_specs=[
            full((N, N)), acol, hblk,
            full((2 * NK, 4 * U)), full((NK, 2 * U, 4 * U)), full((1, 4 * U)),
            full((2 * NK, 2 * U)), full((NK, 2 * U, 2 * U)), full((1, 2 * U)),
            full((NK, 2 * U, 4 * U)), full((NK, 2 * U, 4 * U)), full((1, 4 * U)),
            full((NK, 2 * U, 2 * U)), full((NK, 2 * U, 2 * U)), full((1, 2 * U)),
            full((2 * U, 2)), full((1, 1)),
        ],
        out_specs=[ocol, hblk],
        out_shape=[
            sds((B // 2, 2, N), f32),
            sds((2, B, N, U), f32),
        ],
    )(support,
      jnp.stack([inputs, A1, A2]).reshape(NK, B // 2, 2, N),
      hidden_state.reshape(2, B, N, U),
      wa_ru0, wh_ru0, b_ru0, wa_c0, wh_c0, b_c0,
      wg_ru1, wk_ru1, b_ru1, wg_c1, wk_c1, b_c1,
      wpp, b_proj.reshape(1, 1))

    out = out_p.reshape(B, N)
    hidden = hid.reshape(2, B, N * U)
    return (out, hidden)


# speedup vs baseline: 1.0169x; 1.0169x over previous
import jax
import jax.numpy as jnp
from jax.experimental import pallas as pl

N = 512
U = 64
B = 64
NK = 3


def _prep_kernel(adj_ref, x_ref,
                 wru0_ref, wc0_ref, wru1_ref, wc1_ref,
                 bru0_ref, bc0_ref, bru1_ref, bc1_ref, wp_ref,
                 sup_ref, a1_ref, a2_ref,
                 wa_ru0_ref, wh_ru0_ref, b_ru0_ref,
                 wa_c0_ref, wh_c0_ref, b_c0_ref,
                 wg_ru1_ref, wk_ru1_ref, b_ru1_ref,
                 wg_c1_ref, wk_c1_ref, b_c1_ref, wpp_ref):
    f32 = jnp.float32
    adj = adj_ref[...]
    a = jnp.maximum(adj, adj.T)
    d_col = jnp.sum(a, axis=1, keepdims=True)
    d_row = jnp.sum(a, axis=0, keepdims=True)
    inv_c = jnp.where(d_col > 0, 1.0 / jnp.sqrt(d_col), 0.0)
    inv_r = jnp.where(d_row > 0, 1.0 / jnp.sqrt(d_row), 0.0)
    sup = -(inv_c * a) * inv_r
    sup_ref[...] = sup
    x0 = x_ref[...]
    a1 = jnp.dot(x0, sup, preferred_element_type=f32)
    a1_ref[...] = a1
    a2_ref[...] = 2.0 * jnp.dot(a1, sup, preferred_element_type=f32) - x0

    wa_ru0_ref[...] = jnp.zeros((2 * NK, 4 * U), f32)
    wa_c0_ref[...] = jnp.zeros((2 * NK, 2 * U), f32)
    wh_ru0_ref[...] = jnp.zeros((NK, 2 * U, 4 * U), f32)
    wh_c0_ref[...] = jnp.zeros((NK, 2 * U, 2 * U), f32)
    wg_ru1_ref[...] = jnp.zeros((NK, 2 * U, 4 * U), f32)
    wk_ru1_ref[...] = jnp.zeros((NK, 2 * U, 4 * U), f32)
    wg_c1_ref[...] = jnp.zeros((NK, 2 * U, 2 * U), f32)
    wk_c1_ref[...] = jnp.zeros((NK, 2 * U, 2 * U), f32)
    for k in range(NK):
        wa_ru0_ref[2 * k, 0:U] = wru0_ref[k, 0, 0:U]
        wa_ru0_ref[2 * k, 2 * U:3 * U] = wru0_ref[k, 0, U:2 * U]
        wa_ru0_ref[2 * k + 1, U:2 * U] = wru0_ref[k, 0, 0:U]
        wa_ru0_ref[2 * k + 1, 3 * U:4 * U] = wru0_ref[k, 0, U:2 * U]
        wa_c0_ref[2 * k, 0:U] = wc0_ref[k, 0, :]
        wa_c0_ref[2 * k + 1, U:2 * U] = wc0_ref[k, 0, :]
        whr = wru0_ref[k, 1:U + 1, 0:U]
        whu = wru0_ref[k, 1:U + 1, U:2 * U]
        wh_ru0_ref[k, 0:U, 0:U] = whr
        wh_ru0_ref[k, 0:U, 2 * U:3 * U] = whu
        wh_ru0_ref[k, U:2 * U, U:2 * U] = whr
        wh_ru0_ref[k, U:2 * U, 3 * U:4 * U] = whu
        whc = wc0_ref[k, 1:U + 1, :]
        wh_c0_ref[k, 0:U, 0:U] = whc
        wh_c0_ref[k, U:2 * U, U:2 * U] = whc
        ggr = wru1_ref[k, 0:U, 0:U]
        ggu = wru1_ref[k, 0:U, U:2 * U]
        wg_ru1_ref[k, 0:U, 0:U] = ggr
        wg_ru1_ref[k, 0:U, 2 * U:3 * U] = ggu
        wg_ru1_ref[k, U:2 * U, U:2 * U] = ggr
        wg_ru1_ref[k, U:2 * U, 3 * U:4 * U] = ggu
        kkr = wru1_ref[k, U:2 * U, 0:U]
        kku = wru1_ref[k, U:2 * U, U:2 * U]
        wk_ru1_ref[k, 0:U, 0:U] = kkr
        wk_ru1_ref[k, 0:U, 2 * U:3 * U] = kku
        wk_ru1_ref[k, U:2 * U, U:2 * U] = kkr
        wk_ru1_ref[k, U:2 * U, 3 * U:4 * U] = kku
        ggc = wc1_ref[k, 0:U, :]
        wg_c1_ref[k, 0:U, 0:U] = ggc
        wg_c1_ref[k, U:2 * U, U:2 * U] = ggc
        kkc = wc1_ref[k, U:2 * U, :]
        wk_c1_ref[k, 0:U, 0:U] = kkc
        wk_c1_ref[k, U:2 * U, U:2 * U] = kkc

    b_ru0_ref[0, 0:U] = bru0_ref[0, 0:U]
    b_ru0_ref[0, U:2 * U] = bru0_ref[0, 0:U]
    b_ru0_ref[0, 2 * U:3 * U] = bru0_ref[0, U:2 * U]
    b_ru0_ref[0, 3 * U:4 * U] = bru0_ref[0, U:2 * U]
    b_c0_ref[0, 0:U] = bc0_ref[0, :]
    b_c0_ref[0, U:2 * U] = bc0_ref[0, :]
    b_ru1_ref[0, 0:U] = bru1_ref[0, 0:U]
    b_ru1_ref[0, U:2 * U] = bru1_ref[0, 0:U]
    b_ru1_ref[0, 2 * U:3 * U] = bru1_ref[0, U:2 * U]
    b_ru1_ref[0, 3 * U:4 * U] = bru1_ref[0, U:2 * U]
    b_c1_ref[0, 0:U] = bc1_ref[0, :]
    b_c1_ref[0, U:2 * U] = bc1_ref[0, :]
    wpp_ref[...] = jnp.zeros((2 * U, 2), f32)
    wpp_ref[0:U, 0:1] = wp_ref[...]
    wpp_ref[U:2 * U, 1:2] = wp_ref[...]


def _main_kernel(sup_ref, ac_ref, h_ref,
                 wa_ru0_ref, wh_ru0_ref, b_ru0_ref,
                 wa_c0_ref, wh_c0_ref, b_c0_ref,
                 wg_ru1_ref, wk_ru1_ref, b_ru1_ref,
                 wg_c1_ref, wk_c1_ref, b_c1_ref,
                 wp_ref, bp_ref,
                 out_ref, hid_ref):
    f32 = jnp.float32
    dot = lambda x, y: jnp.dot(x, y, preferred_element_type=f32)
    S = sup_ref[...]
    A = jnp.concatenate([ac_ref[0, 0], ac_ref[1, 0], ac_ref[2, 0]],
                        axis=0).T

    H0 = jnp.concatenate([h_ref[0, 0], h_ref[0, 1]], axis=1)
    ru = b_ru0_ref[...] + dot(A, wa_ru0_ref[...])
    ru += dot(H0, wh_ru0_ref[0])
    H1 = dot(S, H0)
    ru += dot(H1, wh_ru0_ref[1])
    H2 = 2.0 * dot(S, H1) - H0
    ru += dot(H2, wh_ru0_ref[2])
    val = jax.nn.sigmoid(ru)
    r = val[:, :2 * U]
    u = val[:, 2 * U:]
    rh = r * H0
    c = b_c0_ref[...] + dot(A, wa_c0_ref[...])
    c += dot(rh, wh_c0_ref[0])
    R1 = dot(S, rh)
    c += dot(R1, wh_c0_ref[1])
    R2 = 2.0 * dot(S, R1) - rh
    c += dot(R2, wh_c0_ref[2])
    c = jnp.tanh(c)
    h0n = u * H0 + (1.0 - u) * c
    hid_ref[0, 0] = h0n[:, :U]
    hid_ref[0, 1] = h0n[:, U:]

    K0 = jnp.concatenate([h_ref[1, 0], h_ref[1, 1]], axis=1)
    ru1 = b_ru1_ref[...] + dot(h0n, wg_ru1_ref[0]) + dot(K0, wk_ru1_ref[0])
    GK0 = jnp.concatenate([h0n, K0], axis=1)
    GK1 = dot(S, GK0)
    G1 = GK1[:, :2 * U]
    K1 = GK1[:, 2 * U:]
    ru1 += dot(G1, wg_ru1_ref[1]) + dot(K1, wk_ru1_ref[1])
    GK2 = 2.0 * dot(S, GK1) - GK0
    G2 = GK2[:, :2 * U]
    K2 = GK2[:, 2 * U:]
    ru1 += dot(G2, wg_ru1_ref[2]) + dot(K2, wk_ru1_ref[2])
    v1 = jax.nn.sigmoid(ru1)
    r1 = v1[:, :2 * U]
    u1 = v1[:, 2 * U:]
    rh1 = r1 * K0
    c1 = (b_c1_ref[...] + dot(h0n, wg_c1_ref[0]) + dot(G1, wg_c1_ref[1])
          + dot(G2, wg_c1_ref[2]) + dot(rh1, wk_c1_ref[0]))
    Q1 = dot(S, rh1)
    c1 += dot(Q1, wk_c1_ref[1])
    Q2 = 2.0 * dot(S, Q1) - rh1
    c1 += dot(Q2, wk_c1_ref[2])
    c1 = jnp.tanh(c1)
    h1n = u1 * K0 + (1.0 - u1) * c1
    hid_ref[1, 0] = h1n[:, :U]
    hid_ref[1, 1] = h1n[:, U:]
    prj = dot(h1n, wp_ref[...]) + bp_ref[...]
    out_ref[0] = prj.T


def kernel(inputs, hidden_state, adj_mx, W_ru_0, b_ru_0, W_c_0, b_c_0,
           W_ru_1, b_ru_1, W_c_1, b_c_1, W_proj, b_proj):
    f32 = jnp.float32

    wru0 = W_ru_0.reshape(U + 1, NK, 2 * U).transpose(1, 0, 2)
    wc0 = W_c_0.reshape(U + 1, NK, U).transpose(1, 0, 2)
    wru1 = W_ru_1.reshape(2 * U, NK, 2 * U).transpose(1, 0, 2)
    wc1 = W_c_1.reshape(2 * U, NK, U).transpose(1, 0, 2)

    sds = jax.ShapeDtypeStruct
    (support, A1, A2,
     wa_ru0, wh_ru0, b_ru0, wa_c0, wh_c0, b_c0,
     wg_ru1, wk_ru1, b_ru1, wg_c1, wk_c1, b_c1, wpp) = pl.pallas_call(
        _prep_kernel,
        out_shape=[
            sds((N, N), f32), sds((B, N), f32), sds((B, N), f32),
            sds((2 * NK, 4 * U), f32), sds((NK, 2 * U, 4 * U), f32), sds((1, 4 * U), f32),
            sds((2 * NK, 2 * U), f32), sds((NK, 2 * U, 2 * U), f32), sds((1, 2 * U), f32),
            sds((NK, 2 * U, 4 * U), f32), sds((NK, 2 * U, 4 * U), f32), sds((1, 4 * U), f32),
            sds((NK, 2 * U, 2 * U), f32), sds((NK, 2 * U, 2 * U), f32), sds((1, 2 * U), f32),
            sds((2 * U, 2), f32),
        ],
    )(adj_mx, inputs, wru0, wc0, wru1, wc1,
      b_ru_0.reshape(1, 2 * U), b_c_0.reshape(1, U),
      b_ru_1.reshape(1, 2 * U), b_c_1.reshape(1, U), W_proj)

    full = lambda shape: pl.BlockSpec(shape, lambda b: tuple(0 for _ in shape))
    acol = pl.BlockSpec((NK, 1, 2, N), lambda b: (0, b, 0, 0))
    ocol = pl.BlockSpec((1, 2, N), lambda b: (b, 0, 0))
    hblk = pl.BlockSpec((2, 2, N, U), lambda b: (0, b, 0, 0))

    out_p, hid = pl.pallas_call(
        _main_kernel,
        grid=(B // 2,),
        in_specs=[
            full((N, N)), acol, hblk,
            full((2 * NK, 4 * U)), full((NK, 2 * U, 4 * U)), full((1, 4 * U)),
            full((2 * NK, 2 * U)), full((NK, 2 * U, 2 * U)), full((1, 2 * U)),
            full((NK, 2 * U, 4 * U)), full((NK, 2 * U, 4 * U)), full((1, 4 * U)),
            full((NK, 2 * U, 2 * U)), full((NK, 2 * U, 2 * U)), full((1, 2 * U)),
            full((2 * U, 2)), full((1, 1)),
        ],
        out_specs=[ocol, hblk],
        out_shape=[
            sds((B // 2, 2, N), f32),
            sds((2, B, N, U), f32),
        ],
    )(support,
      jnp.stack([inputs, A1, A2]).reshape(NK, B // 2, 2, N),
      hidden_state.reshape(2, B, N, U),
      wa_ru0, wh_ru0, b_ru0, wa_c0, wh_c0, b_c0,
      wg_ru1, wk_ru1, b_ru1, wg_c1, wk_c1, b_c1,
      wpp, b_proj.reshape(1, 1))

    out = out_p.reshape(B, N)
    hidden = hid.reshape(2, B, N * U)
    return (out, hidden)
